# baseline (speedup 1.0000x reference)
import functools

import jax
import jax.numpy as jnp
from jax import lax
from jax.experimental import pallas as pl
from jax.experimental.pallas import tpu as pltpu

B, T, N, D = 2, 2048, 24, 1024
K = 12
KEEP = N - K
TB = 128


def _body(x_ref, wg_ref, out_ref):
    xt = x_ref[...]
    wg = wg_ref[...]
    scores = lax.dot_general(
        xt.reshape(TB * N, D), wg.reshape(D, 1),
        dimension_numbers=(((1,), (0,)), ((), ())),
        preferred_element_type=jnp.float32,
    ).reshape(TB, N)

    st = scores.T
    si = st[:, None, :]
    sj = st[None, :, :]
    ii = lax.broadcasted_iota(jnp.int32, (N, N, TB), 0)
    jj = lax.broadcasted_iota(jnp.int32, (N, N, TB), 1)
    beats = (sj > si) | ((sj == si) & (jj < ii))
    rank = jnp.sum(beats.astype(jnp.int32), axis=1)
    keep = rank < KEEP

    neg = jnp.float32(-1e30)
    masked = jnp.where(keep, st, neg)
    m = jnp.max(masked, axis=0, keepdims=True)
    e = jnp.where(keep, jnp.exp(st - m), 0.0)
    w = (e / jnp.sum(e, axis=0, keepdims=True)).T

    out_ref[...] = jnp.sum(w[:, :, None] * xt, axis=1)


def _tc_call(xf, wg2, F):
    return pl.pallas_call(
        _body,
        grid=(F // TB,),
        in_specs=[
            pl.BlockSpec((TB, N, D), lambda i: (i, 0, 0)),
            pl.BlockSpec((1, D), lambda i: (0, 0)),
        ],
        out_specs=pl.BlockSpec((TB, D), lambda i: (i, 0)),
        out_shape=jax.ShapeDtypeStruct((F, D), jnp.float32),
    )(xf, wg2)


from jax.experimental.pallas import tpu_sc as plsc

_NC, _NS = 2, 16
_NW = _NC * _NS
_C16 = D // 16
_C32 = D // 32


def _bf16_round(v):
    t = v * jnp.float32(65537.0)
    return t - (t - v)


def _bf16_pair(a, b):
    return _bf16_round(a), _bf16_round(b)


def _sc_compute(xslab, wgr, obuf_slot):
    def cbody(c2, accs):
        o = c2 * 32
        w0 = wgr[pl.ds(o, 16)]
        w1 = wgr[pl.ds(o + 16, 16)]
        new = []
        for n in range(N):
            x0r, x1r = _bf16_pair(xslab[n, pl.ds(o, 16)],
                                  xslab[n, pl.ds(o + 16, 16)])
            new.append(accs[n] + x0r * w0 + x1r * w1)
        return tuple(new)

    zero = jnp.zeros((16,), jnp.float32)
    accs = lax.fori_loop(0, _C32, cbody, (zero,) * N)
    s = [jnp.sum(accs[n]) for n in range(N)]

    i16 = lax.iota(jnp.int32, 16)
    s0 = jnp.full((16,), -1e30, jnp.float32)
    s1 = jnp.full((16,), -1e30, jnp.float32)
    for n in range(16):
        s0 = jnp.where(i16 == n, s[n], s0)
    for n in range(16, N):
        s1 = jnp.where(i16 == (n - 16), s[n], s1)

    r0 = jnp.zeros((16,), jnp.int32)
    r1 = jnp.zeros((16,), jnp.int32)
    lt0 = [(jnp.int32(j) < i16) for j in range(N)]
    for j in range(N):
        r0 = r0 + ((s[j] > s0) | ((s[j] == s0) & lt0[j])).astype(jnp.int32)
        r1 = r1 + ((s[j] > s1) | ((s[j] == s1) & (j < i16 + 16))).astype(jnp.int32)
    keep0 = r0 < KEEP
    keep1 = r1 < KEEP

    m = jnp.maximum(jnp.max(s0), jnp.max(s1))
    e0 = jnp.where(keep0, jnp.exp(s0 - m), 0.0)
    e1 = jnp.where(keep1, jnp.exp(s1 - m), 0.0)
    zv = jnp.full((16,), jnp.sum(e0) + jnp.sum(e1), jnp.float32)
    wv0 = e0 / zv
    wv1 = e1 / zv
    ws = [wv0[n] for n in range(16)] + [wv1[n - 16] for n in range(16, N)]

    def obody(c, _):
        o = c * 16
        acc = ws[0] * xslab[0, pl.ds(o, 16)]
        for n in range(1, N):
            acc = acc + ws[n] * xslab[n, pl.ds(o, 16)]
        obuf_slot[pl.ds(o, 16)] = acc
        return 0

    lax.fori_loop(0, _C16, obody, 0)


def _sc_call(xf, wg, F):
    fpw = F // _NW
    ng = fpw // 2
    mesh = plsc.VectorSubcoreMesh(core_axis_name="c", subcore_axis_name="s",
                                  num_cores=_NC, num_subcores=_NS)

    @functools.partial(
        pl.kernel,
        out_type=jax.ShapeDtypeStruct((F, D), jnp.float32),
        mesh=mesh,
        compiler_params=pltpu.CompilerParams(needs_layout_passes=False),
        scratch_types=[
            pltpu.VMEM((N, D), jnp.float32),
            pltpu.VMEM((N, D), jnp.float32),
            pltpu.VMEM((D,), jnp.float32),
            pltpu.VMEM((D,), jnp.float32),
            pltpu.VMEM((D,), jnp.float32),
            pltpu.SemaphoreType.DMA,
            pltpu.SemaphoreType.DMA,
            pltpu.SemaphoreType.DMA,
            pltpu.SemaphoreType.DMA,
        ],
    )
    def k(x_hbm, wg_hbm, out_hbm, xbuf0, xbuf1, wgr, obuf0, obuf1,
          sem_i0, sem_i1, sem_o0, sem_o1):
        wid = lax.axis_index("s") * _NC + lax.axis_index("c")
        base = wid * fpw

        pltpu.sync_copy(wg_hbm, wgr)

        def rbody(c2, _):
            o = c2 * 32
            w0r, w1r = _bf16_pair(wgr[pl.ds(o, 16)], wgr[pl.ds(o + 16, 16)])
            wgr[pl.ds(o, 16)] = w0r
            wgr[pl.ds(o + 16, 16)] = w1r
            return 0

        lax.fori_loop(0, _C32, rbody, 0)

        sem_i = (sem_i0, sem_i1)
        sem_o = (sem_o0, sem_o1)
        xbufs = (xbuf0, xbuf1)
        obufs = (obuf0, obuf1)

        def in_copy(f, slot):
            return pltpu.make_async_copy(x_hbm.at[base + f], xbufs[slot],
                                         sem_i[slot])

        def out_copy(f, slot):
            return pltpu.make_async_copy(obufs[slot], out_hbm.at[base + f],
                                         sem_o[slot])

        in_copy(0, 0).start()

        def gbody(g, _):
            f = 2 * g
            for slot in range(2):
                nxt = f + slot + 1

                @pl.when(nxt < fpw)
                def _():
                    in_copy(nxt, 1 - slot if slot == 0 else 0).start()

                in_copy(f + slot, slot).wait()
                @pl.when(g > 0)
                def _():
                    out_copy(f + slot - 2, slot).wait()

                _sc_compute(xbufs[slot], wgr, obufs[slot])
                out_copy(f + slot, slot).start()
            return 0

        lax.fori_loop(0, ng, gbody, 0)
        out_copy(fpw - 2, 0).wait()
        out_copy(fpw - 1, 1).wait()

    return k(xf, wg)


@jax.jit
def kernel(x, w_g):
    F = B * T
    xf = x.reshape(F, N, D)
    out = _sc_call(xf, w_g, F)
    return out.reshape(B, T, D)


# speedup vs baseline: 5.2019x; 5.2019x over previous
import functools

import jax
import jax.numpy as jnp
from jax import lax
from jax.experimental import pallas as pl
from jax.experimental.pallas import tpu as pltpu

B, T, N, D = 2, 2048, 24, 1024
K = 12
KEEP = N - K
TB = 128


def _body(x_ref, wg_ref, out_ref):
    xt = x_ref[...]
    wg = wg_ref[...]
    scores = lax.dot_general(
        xt.reshape(TB * N, D), wg.reshape(D, 1),
        dimension_numbers=(((1,), (0,)), ((), ())),
        preferred_element_type=jnp.float32,
    ).reshape(TB, N)

    st = scores.T
    si = st[:, None, :]
    sj = st[None, :, :]
    ii = lax.broadcasted_iota(jnp.int32, (N, N, TB), 0)
    jj = lax.broadcasted_iota(jnp.int32, (N, N, TB), 1)
    beats = (sj > si) | ((sj == si) & (jj < ii))
    rank = jnp.sum(beats.astype(jnp.int32), axis=1)
    keep = rank < KEEP

    neg = jnp.float32(-1e30)
    masked = jnp.where(keep, st, neg)
    m = jnp.max(masked, axis=0, keepdims=True)
    e = jnp.where(keep, jnp.exp(st - m), 0.0)
    w = (e / jnp.sum(e, axis=0, keepdims=True)).T

    out_ref[...] = jnp.sum(w[:, :, None] * xt, axis=1)


def _tc_call(xf, wg2, F):
    return pl.pallas_call(
        _body,
        grid=(F // TB,),
        in_specs=[
            pl.BlockSpec((TB, N, D), lambda i: (i, 0, 0)),
            pl.BlockSpec((1, D), lambda i: (0, 0)),
        ],
        out_specs=pl.BlockSpec((TB, D), lambda i: (i, 0)),
        out_shape=jax.ShapeDtypeStruct((F, D), jnp.float32),
    )(xf, wg2)


from jax.experimental.pallas import tpu_sc as plsc

_NC, _NS = 2, 16
_NW = _NC * _NS
_C16 = D // 16
_C32 = D // 32


def _bf16_round(v):
    t = v * jnp.float32(65537.0)
    return t - (t - v)


def _bf16_pair(a, b):
    return _bf16_round(a), _bf16_round(b)


def _sc_compute(xslab, wgr, obuf_slot):
    def cbody(c2, accs):
        o = c2 * 32
        w0 = wgr[pl.ds(o, 16)]
        w1 = wgr[pl.ds(o + 16, 16)]
        new = []
        for n in range(N):
            x0r, x1r = _bf16_pair(xslab[n, pl.ds(o, 16)],
                                  xslab[n, pl.ds(o + 16, 16)])
            new.append(accs[n] + x0r * w0 + x1r * w1)
        return tuple(new)

    zero = jnp.zeros((16,), jnp.float32)
    accs = lax.fori_loop(0, _C32, cbody, (zero,) * N)
    s = [jnp.sum(accs[n]) for n in range(N)]

    i16 = lax.iota(jnp.int32, 16)
    s0 = jnp.full((16,), -1e30, jnp.float32)
    s1 = jnp.full((16,), -1e30, jnp.float32)
    for n in range(16):
        s0 = jnp.where(i16 == n, s[n], s0)
    for n in range(16, N):
        s1 = jnp.where(i16 == (n - 16), s[n], s1)

    r0 = jnp.zeros((16,), jnp.int32)
    r1 = jnp.zeros((16,), jnp.int32)
    lt0 = [(jnp.int32(j) < i16) for j in range(N)]
    for j in range(N):
        r0 = r0 + ((s[j] > s0) | ((s[j] == s0) & lt0[j])).astype(jnp.int32)
        r1 = r1 + ((s[j] > s1) | ((s[j] == s1) & (j < i16 + 16))).astype(jnp.int32)
    keep0 = r0 < KEEP
    keep1 = r1 < KEEP

    m = jnp.maximum(jnp.max(s0), jnp.max(s1))
    e0 = jnp.where(keep0, jnp.exp(s0 - m), 0.0)
    e1 = jnp.where(keep1, jnp.exp(s1 - m), 0.0)
    zv = jnp.full((16,), jnp.sum(e0) + jnp.sum(e1), jnp.float32)
    wv0 = e0 / zv
    wv1 = e1 / zv
    ws = [wv0[n] for n in range(16)] + [wv1[n - 16] for n in range(16, N)]

    def obody(c, _):
        o = c * 16
        acc = ws[0] * xslab[0, pl.ds(o, 16)]
        for n in range(1, N):
            acc = acc + ws[n] * xslab[n, pl.ds(o, 16)]
        obuf_slot[pl.ds(o, 16)] = acc
        return 0

    lax.fori_loop(0, _C16, obody, 0)


def _sc_call(xf, wg, f_sc, f_base):
    fpw = f_sc // _NW
    ng = fpw // 2
    mesh = plsc.VectorSubcoreMesh(core_axis_name="c", subcore_axis_name="s",
                                  num_cores=_NC, num_subcores=_NS)

    @functools.partial(
        pl.kernel,
        out_type=jax.ShapeDtypeStruct((f_sc, D), jnp.float32),
        mesh=mesh,
        compiler_params=pltpu.CompilerParams(needs_layout_passes=False),
        scratch_types=[
            pltpu.VMEM((N, D), jnp.float32),
            pltpu.VMEM((N, D), jnp.float32),
            pltpu.VMEM((D,), jnp.float32),
            pltpu.VMEM((D,), jnp.float32),
            pltpu.VMEM((D,), jnp.float32),
            pltpu.SemaphoreType.DMA,
            pltpu.SemaphoreType.DMA,
            pltpu.SemaphoreType.DMA,
            pltpu.SemaphoreType.DMA,
        ],
    )
    def k(x_hbm, wg_hbm, out_hbm, xbuf0, xbuf1, wgr, obuf0, obuf1,
          sem_i0, sem_i1, sem_o0, sem_o1):
        wid = lax.axis_index("s") * _NC + lax.axis_index("c")
        base = f_base + wid * fpw
        obase = wid * fpw

        pltpu.sync_copy(wg_hbm, wgr)

        def rbody(c2, _):
            o = c2 * 32
            w0r, w1r = _bf16_pair(wgr[pl.ds(o, 16)], wgr[pl.ds(o + 16, 16)])
            wgr[pl.ds(o, 16)] = w0r
            wgr[pl.ds(o + 16, 16)] = w1r
            return 0

        lax.fori_loop(0, _C32, rbody, 0)

        sem_i = (sem_i0, sem_i1)
        sem_o = (sem_o0, sem_o1)
        xbufs = (xbuf0, xbuf1)
        obufs = (obuf0, obuf1)

        def in_copy(f, slot):
            return pltpu.make_async_copy(x_hbm.at[base + f], xbufs[slot],
                                         sem_i[slot])

        def out_copy(f, slot):
            return pltpu.make_async_copy(obufs[slot], out_hbm.at[obase + f],
                                         sem_o[slot])

        in_copy(0, 0).start()

        def gbody(g, _):
            f = 2 * g
            for slot in range(2):
                nxt = f + slot + 1

                @pl.when(nxt < fpw)
                def _():
                    in_copy(nxt, 1 - slot if slot == 0 else 0).start()

                in_copy(f + slot, slot).wait()
                @pl.when(g > 0)
                def _():
                    out_copy(f + slot - 2, slot).wait()

                _sc_compute(xbufs[slot], wgr, obufs[slot])
                out_copy(f + slot, slot).start()
            return 0

        lax.fori_loop(0, ng, gbody, 0)
        out_copy(fpw - 2, 0).wait()
        out_copy(fpw - 1, 1).wait()

    return k(xf, wg)


F_SC = 512
F_TC = B * T - F_SC


@jax.jit
def kernel(x, w_g):
    F = B * T
    xf = x.reshape(F, N, D)
    out_sc = _sc_call(xf, w_g, F_SC, F_TC)
    out_tc = _tc_call(xf, w_g.reshape(1, D), F_TC)
    out = jnp.concatenate([out_tc, out_sc], axis=0)
    return out.reshape(B, T, D)


# speedup vs baseline: 6.1827x; 1.1885x over previous
import functools

import jax
import jax.numpy as jnp
from jax import lax
from jax.experimental import pallas as pl
from jax.experimental.pallas import tpu as pltpu

B, T, N, D = 2, 2048, 24, 1024
K = 12
KEEP = N - K
TB = 256


def _body(x_ref, wg_ref, out_ref):
    xt = x_ref[...]
    wg = wg_ref[...]
    scores = lax.dot_general(
        xt.reshape(TB * N, D), wg.reshape(D, 1),
        dimension_numbers=(((1,), (0,)), ((), ())),
        preferred_element_type=jnp.float32,
    ).reshape(TB, N)

    st = scores.T
    si = st[:, None, :]
    sj = st[None, :, :]
    ii = lax.broadcasted_iota(jnp.int32, (N, N, TB), 0)
    jj = lax.broadcasted_iota(jnp.int32, (N, N, TB), 1)
    beats = (sj > si) | ((sj == si) & (jj < ii))
    rank = jnp.sum(beats.astype(jnp.int32), axis=1)
    keep = rank < KEEP

    neg = jnp.float32(-1e30)
    masked = jnp.where(keep, st, neg)
    m = jnp.max(masked, axis=0, keepdims=True)
    e = jnp.where(keep, jnp.exp(st - m), 0.0)
    w = (e / jnp.sum(e, axis=0, keepdims=True)).T

    out_ref[...] = jnp.sum(w[:, :, None] * xt, axis=1)


def _tc_call(xf, wg2, F):
    return pl.pallas_call(
        _body,
        grid=(F // TB,),
        in_specs=[
            pl.BlockSpec((TB, N, D), lambda i: (i, 0, 0)),
            pl.BlockSpec((1, D), lambda i: (0, 0)),
        ],
        out_specs=pl.BlockSpec((TB, D), lambda i: (i, 0)),
        out_shape=jax.ShapeDtypeStruct((F, D), jnp.float32),
    )(xf, wg2)


from jax.experimental.pallas import tpu_sc as plsc

_NC, _NS = 2, 16
_NW = _NC * _NS
_C16 = D // 16
_C32 = D // 32


def _bf16_round(v):
    t = v * jnp.float32(65537.0)
    return t - (t - v)


def _bf16_pair(a, b):
    return _bf16_round(a), _bf16_round(b)


def _sc_compute(xslab, wgr, obuf_slot):
    def cbody(c2, accs):
        o = c2 * 32
        w0 = wgr[pl.ds(o, 16)]
        w1 = wgr[pl.ds(o + 16, 16)]
        new = []
        for n in range(N):
            x0r, x1r = _bf16_pair(xslab[n, pl.ds(o, 16)],
                                  xslab[n, pl.ds(o + 16, 16)])
            new.append(accs[n] + x0r * w0 + x1r * w1)
        return tuple(new)

    zero = jnp.zeros((16,), jnp.float32)
    accs = lax.fori_loop(0, _C32, cbody, (zero,) * N)
    s = [jnp.sum(accs[n]) for n in range(N)]

    i16 = lax.iota(jnp.int32, 16)
    s0 = jnp.full((16,), -1e30, jnp.float32)
    s1 = jnp.full((16,), -1e30, jnp.float32)
    for n in range(16):
        s0 = jnp.where(i16 == n, s[n], s0)
    for n in range(16, N):
        s1 = jnp.where(i16 == (n - 16), s[n], s1)

    r0 = jnp.zeros((16,), jnp.int32)
    r1 = jnp.zeros((16,), jnp.int32)
    lt0 = [(jnp.int32(j) < i16) for j in range(N)]
    for j in range(N):
        r0 = r0 + ((s[j] > s0) | ((s[j] == s0) & lt0[j])).astype(jnp.int32)
        r1 = r1 + ((s[j] > s1) | ((s[j] == s1) & (j < i16 + 16))).astype(jnp.int32)
    keep0 = r0 < KEEP
    keep1 = r1 < KEEP

    m = jnp.maximum(jnp.max(s0), jnp.max(s1))
    e0 = jnp.where(keep0, jnp.exp(s0 - m), 0.0)
    e1 = jnp.where(keep1, jnp.exp(s1 - m), 0.0)
    zv = jnp.full((16,), jnp.sum(e0) + jnp.sum(e1), jnp.float32)
    wv0 = e0 / zv
    wv1 = e1 / zv
    ws = [wv0[n] for n in range(16)] + [wv1[n - 16] for n in range(16, N)]

    def obody(c, _):
        o = c * 16
        acc = ws[0] * xslab[0, pl.ds(o, 16)]
        for n in range(1, N):
            acc = acc + ws[n] * xslab[n, pl.ds(o, 16)]
        obuf_slot[pl.ds(o, 16)] = acc
        return 0

    lax.fori_loop(0, _C16, obody, 0)


def _sc_call(xf, wg, f_sc, f_base):
    fpw = f_sc // _NW
    ng = fpw // 2
    mesh = plsc.VectorSubcoreMesh(core_axis_name="c", subcore_axis_name="s",
                                  num_cores=_NC, num_subcores=_NS)

    @functools.partial(
        pl.kernel,
        out_type=jax.ShapeDtypeStruct((f_sc, D), jnp.float32),
        mesh=mesh,
        compiler_params=pltpu.CompilerParams(needs_layout_passes=False),
        scratch_types=[
            pltpu.VMEM((N, D), jnp.float32),
            pltpu.VMEM((N, D), jnp.float32),
            pltpu.VMEM((D,), jnp.float32),
            pltpu.VMEM((D,), jnp.float32),
            pltpu.VMEM((D,), jnp.float32),
            pltpu.SemaphoreType.DMA,
            pltpu.SemaphoreType.DMA,
            pltpu.SemaphoreType.DMA,
            pltpu.SemaphoreType.DMA,
        ],
    )
    def k(x_hbm, wg_hbm, out_hbm, xbuf0, xbuf1, wgr, obuf0, obuf1,
          sem_i0, sem_i1, sem_o0, sem_o1):
        wid = lax.axis_index("s") * _NC + lax.axis_index("c")
        base = f_base + wid * fpw
        obase = wid * fpw

        pltpu.sync_copy(wg_hbm, wgr)

        def rbody(c2, _):
            o = c2 * 32
            w0r, w1r = _bf16_pair(wgr[pl.ds(o, 16)], wgr[pl.ds(o + 16, 16)])
            wgr[pl.ds(o, 16)] = w0r
            wgr[pl.ds(o + 16, 16)] = w1r
            return 0

        lax.fori_loop(0, _C32, rbody, 0)

        sem_i = (sem_i0, sem_i1)
        sem_o = (sem_o0, sem_o1)
        xbufs = (xbuf0, xbuf1)
        obufs = (obuf0, obuf1)

        def in_copy(f, slot):
            return pltpu.make_async_copy(x_hbm.at[base + f], xbufs[slot],
                                         sem_i[slot])

        def out_copy(f, slot):
            return pltpu.make_async_copy(obufs[slot], out_hbm.at[obase + f],
                                         sem_o[slot])

        in_copy(0, 0).start()

        def gbody(g, _):
            f = 2 * g
            for slot in range(2):
                nxt = f + slot + 1

                @pl.when(nxt < fpw)
                def _():
                    in_copy(nxt, 1 - slot if slot == 0 else 0).start()

                in_copy(f + slot, slot).wait()
                @pl.when(g > 0)
                def _():
                    out_copy(f + slot - 2, slot).wait()

                _sc_compute(xbufs[slot], wgr, obufs[slot])
                out_copy(f + slot, slot).start()
            return 0

        lax.fori_loop(0, ng, gbody, 0)
        out_copy(fpw - 2, 0).wait()
        out_copy(fpw - 1, 1).wait()

    return k(xf, wg)


@jax.jit
def kernel(x, w_g):
    F = B * T
    xf = x.reshape(F, N, D)
    out = _tc_call(xf, w_g.reshape(1, D), F)
    return out.reshape(B, T, D)
